# baseline (speedup 1.0000x reference)
import functools

import jax
import jax.numpy as jnp
from jax import lax
from jax.experimental import pallas as pl
from jax.experimental.pallas import tpu as pltpu
from jax.experimental.pallas import tpu_sc as plsc

NC = 2
NS = 16
NW = NC * NS
LANES = 16

VOCAB = 1000000
EMBED_DIM = 32
BATCH = 16384
HIST = 50

GBATCH = 128
BBLK = BATCH // GBATCH
QB = BBLK // NW
DR = EMBED_DIM // 8
RPITCH = EMBED_DIM + 1


def _transpose_block(rows, tbuf_r_slices, row_idx):
    for r in range(DR):
        for s2 in range(4):
            vals = []
            for ds_ in range(2):
                d = r * 8 + s2 * 2 + ds_
                col_idx = jnp.full((LANES,), d, jnp.int32)
                for lg in range(8):
                    vals.append(
                        plsc.load_gather(rows, [row_idx[lg], col_idx])
                    )
            for ds_ in range(2):
                s = s2 * 2 + ds_
                for lg in range(8):
                    tbuf_r_slices[r][s, pl.ds(lg * LANES, LANES)] = (
                        vals[ds_ * 8 + lg]
                    )


def _body(x_hbm, table_hbm, out_hbm, idx_v, rows_v, tbuf_v, gsem, wsem):
    wid = lax.axis_index("s") * NC + lax.axis_index("c")
    cb0 = wid * QB

    pltpu.sync_copy(x_hbm.at[:, pl.ds(cb0, QB)], idx_v)

    iota = lax.iota(jnp.int32, LANES)
    row_idx = [iota + lg * LANES for lg in range(8)]

    def start_gather(h, p):
        for q in range(QB):
            pltpu.async_copy(
                table_hbm.at[idx_v.at[h, q]], rows_v.at[p, q], gsem
            )

    def wait_gather(h, p):
        for q in range(QB):
            pltpu.make_async_copy(
                table_hbm.at[idx_v.at[h, q]], rows_v.at[p, q], gsem
            ).wait()

    def start_write(h, p):
        for r in range(DR):
            pltpu.async_copy(
                tbuf_v.at[p, r], out_hbm.at[h, r, pl.ds(cb0, QB)], wsem
            )

    def wait_write(h, p):
        for r in range(DR):
            pltpu.make_async_copy(
                tbuf_v.at[p, r], out_hbm.at[h, r, pl.ds(cb0, QB)], wsem
            ).wait()

    def transpose(p):
        @plsc.parallel_loop(0, QB)
        def _(q):
            _transpose_block(
                rows_v.at[p, q],
                [tbuf_v.at[p, r, q] for r in range(DR)],
                row_idx,
            )

    start_gather(0, 0)

    def step(h, carry):
        p = lax.rem(h, 2)
        wait_gather(h, p)

        @pl.when(h < HIST - 1)
        def _():
            start_gather(h + 1, 1 - p)

        @pl.when(h >= 2)
        def _():
            wait_write(h - 2, p)

        transpose(p)
        start_write(h, p)
        return carry

    lax.fori_loop(0, HIST, step, 0)
    wait_write(HIST - 2, 0)
    wait_write(HIST - 1, 1)


@functools.partial(
    pl.kernel,
    out_type=jax.ShapeDtypeStruct((HIST, DR, BBLK, 8, GBATCH), jnp.float32),
    mesh=plsc.VectorSubcoreMesh(
        core_axis_name="c", subcore_axis_name="s", num_cores=NC, num_subcores=NS
    ),
    scratch_types=[
        pltpu.VMEM((HIST, QB, GBATCH), jnp.int32),
        pltpu.VMEM((2, QB, GBATCH, RPITCH), jnp.float32),
        pltpu.VMEM((2, DR, QB, 8, GBATCH), jnp.float32),
        pltpu.SemaphoreType.DMA,
        pltpu.SemaphoreType.DMA,
    ],
    compiler_params=pltpu.CompilerParams(
        use_tc_tiling_on_sc=False, needs_layout_passes=False
    ),
)
def _gather_kernel(x_hbm, table_hbm, out_hbm, idx_v, rows_v, tbuf_v, g, w):
    _body(x_hbm, table_hbm, out_hbm, idx_v, rows_v, tbuf_v, g, w)


def kernel(x, table):
    xt = jnp.transpose(x).astype(jnp.int32).reshape(HIST, BBLK, GBATCH)
    tpad = jnp.pad(table, ((0, 0), (0, RPITCH - EMBED_DIM)))
    res = _gather_kernel(xt, tpad)
    return res.transpose(2, 4, 0, 1, 3).reshape(BATCH, HIST, EMBED_DIM)


# speedup vs baseline: 1.0933x; 1.0933x over previous
import functools

import jax
import jax.numpy as jnp
from jax import lax
from jax.experimental import pallas as pl
from jax.experimental.pallas import tpu as pltpu
from jax.experimental.pallas import tpu_sc as plsc

NC = 2
NS = 16
NW = NC * NS
LANES = 16

VOCAB = 1000000
EMBED_DIM = 32
BATCH = 16384
HIST = 50

GBATCH = 128
BBLK = BATCH // GBATCH
QB = BBLK // NW
DR = EMBED_DIM // 8


def _transpose_block(rows, tbuf_r_slices, row_idx):
    for r in range(DR):
        for s2 in range(4):
            vals = []
            for ds_ in range(2):
                d = r * 8 + s2 * 2 + ds_
                col_idx = jnp.full((LANES,), d, jnp.int32)
                for lg in range(8):
                    vals.append(
                        plsc.load_gather(rows, [row_idx[lg], col_idx])
                    )
            for ds_ in range(2):
                s = s2 * 2 + ds_
                for lg in range(8):
                    tbuf_r_slices[r][s, pl.ds(lg * LANES, LANES)] = (
                        vals[ds_ * 8 + lg]
                    )


def _body(x_hbm, table_hbm, out_hbm, idx_v, rows_v, tbuf_v, gsem, wsem):
    wid = lax.axis_index("s") * NC + lax.axis_index("c")
    cb0 = wid * QB

    pltpu.sync_copy(x_hbm.at[:, pl.ds(cb0, QB)], idx_v)

    iota = lax.iota(jnp.int32, LANES)
    row_idx = [iota + lg * LANES for lg in range(8)]

    def start_gather(h, p):
        for q in range(QB):
            pltpu.async_copy(
                table_hbm.at[idx_v.at[h, q]], rows_v.at[p, q], gsem
            )

    def wait_gather(h, p):
        for q in range(QB):
            pltpu.make_async_copy(
                table_hbm.at[idx_v.at[h, q]], rows_v.at[p, q], gsem
            ).wait()

    def start_write(h, p):
        for r in range(DR):
            pltpu.async_copy(
                tbuf_v.at[p, r], out_hbm.at[h, r, pl.ds(cb0, QB)], wsem
            )

    def wait_write(h, p):
        for r in range(DR):
            pltpu.make_async_copy(
                tbuf_v.at[p, r], out_hbm.at[h, r, pl.ds(cb0, QB)], wsem
            ).wait()

    def transpose(p):
        @plsc.parallel_loop(0, QB)
        def _(q):
            _transpose_block(
                rows_v.at[p, q],
                [tbuf_v.at[p, r, q] for r in range(DR)],
                row_idx,
            )

    start_gather(0, 0)

    def step(h, carry):
        p = lax.rem(h, 2)
        wait_gather(h, p)

        @pl.when(h < HIST - 1)
        def _():
            start_gather(h + 1, 1 - p)

        @pl.when(h >= 2)
        def _():
            wait_write(h - 2, p)

        transpose(p)
        start_write(h, p)
        return carry

    lax.fori_loop(0, HIST, step, 0)
    wait_write(HIST - 2, 0)
    wait_write(HIST - 1, 1)


@functools.partial(
    pl.kernel,
    out_type=jax.ShapeDtypeStruct((HIST, DR, BBLK, 8, GBATCH), jnp.float32),
    mesh=plsc.VectorSubcoreMesh(
        core_axis_name="c", subcore_axis_name="s", num_cores=NC, num_subcores=NS
    ),
    scratch_types=[
        pltpu.VMEM((HIST, QB, GBATCH), jnp.int32),
        pltpu.VMEM((2, QB, GBATCH, EMBED_DIM), jnp.float32),
        pltpu.VMEM((2, DR, QB, 8, GBATCH), jnp.float32),
        pltpu.SemaphoreType.DMA,
        pltpu.SemaphoreType.DMA,
    ],
    compiler_params=pltpu.CompilerParams(
        use_tc_tiling_on_sc=False, needs_layout_passes=False
    ),
)
def _gather_kernel(x_hbm, table_hbm, out_hbm, idx_v, rows_v, tbuf_v, g, w):
    _body(x_hbm, table_hbm, out_hbm, idx_v, rows_v, tbuf_v, g, w)


def kernel(x, table):
    xt = jnp.transpose(x).astype(jnp.int32).reshape(HIST, BBLK, GBATCH)
    res = _gather_kernel(xt, table)
    return res.transpose(2, 4, 0, 1, 3).reshape(BATCH, HIST, EMBED_DIM)
